# Initial kernel scaffold; baseline (speedup 1.0000x reference)
import jax
import jax.numpy as jnp
from jax.experimental import pallas as pl


def kernel(x, lens, table, W, b):
    raise NotImplementedError("write your pallas kernel here")


# speedup vs baseline: 1.0241x; 1.0241x over previous
import functools

import jax
import jax.numpy as jnp
from jax import lax
from jax.experimental import pallas as pl
from jax.experimental.pallas import tpu as pltpu
from jax.experimental.pallas import tpu_sc as plsc

B, L, V, D, O = 4096, 50, 1000000, 64, 1

NC, NS = 2, 16
NW = NC * NS
BPW = B // NW
NB = 16
R = NB * L
NCH = BPW // NB

_mesh = plsc.VectorSubcoreMesh(core_axis_name="c", subcore_axis_name="s")


@functools.partial(
    pl.kernel,
    out_type=jax.ShapeDtypeStruct((B,), jnp.float32),
    mesh=_mesh,
    scratch_types=[
        pltpu.VMEM((BPW * L,), jnp.int32),
        pltpu.VMEM((BPW,), jnp.int32),
        pltpu.VMEM((D,), jnp.float32),
        pltpu.VMEM((16,), jnp.float32),
        pltpu.VMEM((R, D), jnp.float32),
        pltpu.VMEM((R, D), jnp.float32),
        pltpu.VMEM((BPW,), jnp.float32),
        pltpu.SemaphoreType.DMA,
        pltpu.SemaphoreType.DMA,
    ],
    compiler_params=pltpu.CompilerParams(
        needs_layout_passes=False, use_tc_tiling_on_sc=False),
)
def _sc_embed_linear(table_h, xf_h, lens_h, w_h, b_h, out_h,
                     x_v, lens_v, w_v, b_v, g0, g1, out_v, sem0, sem1):
    wid = lax.axis_index("s") * NC + lax.axis_index("c")
    bbase = wid * BPW

    pltpu.sync_copy(xf_h.at[pl.ds(bbase * L, BPW * L)], x_v)
    pltpu.sync_copy(lens_h.at[pl.ds(bbase, BPW)], lens_v)
    pltpu.sync_copy(w_h, w_v)
    pltpu.sync_copy(b_h, b_v)

    w0 = w_v[pl.ds(0, 16)]
    w1 = w_v[pl.ds(16, 16)]
    w2 = w_v[pl.ds(32, 16)]
    w3 = w_v[pl.ds(48, 16)]
    bvec = b_v[...]
    zero = jnp.zeros((16,), jnp.float32)
    lane = lax.iota(jnp.int32, 16)

    bufs = (g0, g1)
    sems = (sem0, sem1)
    copies = [None, None]
    copies[0] = pltpu.async_copy(
        table_h.at[x_v.at[pl.ds(0, R)]], bufs[0], sems[0])

    for c in range(NCH):
        if c + 1 < NCH:
            nxt = (c + 1) % 2
            copies[nxt] = pltpu.async_copy(
                table_h.at[x_v.at[pl.ds((c + 1) * R, R)]], bufs[nxt], sems[nxt])
        cur = c % 2
        copies[cur].wait()
        g = bufs[cur]
        lvec = lens_v[pl.ds(c * NB, 16)]
        outvec = zero
        for bi in range(NB):
            ln = lvec[bi]
            base = bi * L

            def row_body(j, acc, g=g, base=base):
                a0, a1, a2, a3 = acc
                r = base + j
                return (a0 + g[r, pl.ds(0, 16)],
                        a1 + g[r, pl.ds(16, 16)],
                        a2 + g[r, pl.ds(32, 16)],
                        a3 + g[r, pl.ds(48, 16)])

            a0, a1, a2, a3 = lax.fori_loop(
                0, ln, row_body, (zero, zero, zero, zero))
            s = a0 * w0 + a1 * w1 + a2 * w2 + a3 * w3
            tot = jnp.sum(s)
            outvec = jnp.where(lane == bi, tot, outvec)
        outvec = outvec / lvec.astype(jnp.float32) + bvec
        out_v[pl.ds(c * NB, 16)] = outvec

    pltpu.sync_copy(out_v, out_h.at[pl.ds(bbase, BPW)])


def kernel(x, lens, table, W, b):
    xf = x.astype(jnp.int32).reshape(B * L)
    lens32 = lens.astype(jnp.int32)
    wv = W.reshape(D).astype(jnp.float32)
    bv = jnp.broadcast_to(b.astype(jnp.float32), (16,))
    out = _sc_embed_linear(table, xf, lens32, wv, bv)
    return out.reshape(B, O)
